# baseline (speedup 1.0000x reference)
import numpy as np
import jax
import jax.numpy as jnp
from jax import lax
from jax.experimental import pallas as pl
from jax.experimental.pallas import tpu as pltpu

_TIMESTEPS = 500
_K = 4
_B = 16
_N = 16384
_ROWS = (_N * _K) // 128


def _cosine_alphas_bar_host(timesteps, s=0.008):
    steps = timesteps + 2
    x = np.linspace(0, steps, steps)
    ac = np.cos(0.5 * np.pi * ((x / steps) + s) / (1 + s)) ** 2
    ac = ac / ac[0]
    alphas_step = ac[1:] / ac[:-1]
    betas = np.clip(1.0 - alphas_step, 0.0, 0.9999)
    return np.exp(np.cumsum(np.log(1.0 - betas))).astype(np.float32)


_ALPHAS_BAR = _cosine_alphas_bar_host(_TIMESTEPS)
_TINY = np.float32(np.finfo(np.float32).tiny)


def _group_rot(v, d):
    if d == 0:
        return v
    lane = lax.broadcasted_iota(jnp.int32, v.shape, 1)
    c = lane & 3
    fwd = jnp.roll(v, -d, 1)
    bwd = jnp.roll(v, 4 - d, 1)
    return jnp.where(c < 4 - d, fwd, bwd)


def _fused_kernel(t_ref, tab_ref, b_ref, x_ref, prob_ref, noise_ref):
    shape = (_ROWS, 128)
    lane = lax.broadcasted_iota(jnp.int32, shape, 1)
    row = lax.broadcasted_iota(jnp.int32, shape, 0)
    m = row * 128 + lane
    c = lane & 3

    def rotl(v, d):
        return lax.shift_left(v, jnp.int32(d)) | lax.shift_right_logical(
            v, jnp.int32(32 - d))

    def rounds(x0, x1, rots):
        for r in rots:
            x0 = x0 + x1
            x1 = rotl(x1, r)
            x1 = x0 ^ x1
        return x0, x1

    ks0 = jnp.int32(0)
    ks1 = jnp.int32(1)
    ks2 = jnp.int32(0x1BD11BDB)
    r13 = (13, 15, 26, 6)
    r17 = (17, 29, 16, 24)
    x0 = jnp.zeros(shape, jnp.int32) + ks0
    x1 = m + ks1
    x0, x1 = rounds(x0, x1, r13)
    x0, x1 = x0 + ks1, x1 + (ks2 + jnp.int32(1))
    x0, x1 = rounds(x0, x1, r17)
    x0, x1 = x0 + ks2, x1 + (ks0 + jnp.int32(2))
    x0, x1 = rounds(x0, x1, r13)
    x0, x1 = x0 + ks0, x1 + (ks1 + jnp.int32(3))
    x0, x1 = rounds(x0, x1, r17)
    x0, x1 = x0 + ks1, x1 + (ks2 + jnp.int32(4))
    x0, x1 = rounds(x0, x1, r13)
    x0, x1 = x0 + ks2, x1 + (ks0 + jnp.int32(5))
    bits = x0 ^ x1

    fb = lax.shift_right_logical(bits, jnp.int32(9)) | jnp.int32(0x3F800000)
    f = lax.bitcast_convert_type(fb, jnp.float32) - jnp.float32(1.0)
    u = jnp.maximum(jnp.float32(_TINY),
                    f * (jnp.float32(1.0) - _TINY) + _TINY)
    gum = -jnp.log(-jnp.log(u))

    alphas = []
    for g in range(_B):
        tg = t_ref[g, 0]
        tf = tg.astype(jnp.float32) / jnp.float32(_TIMESTEPS)
        tidx = (tf * jnp.float32(_TIMESTEPS) + jnp.float32(0.5)).astype(jnp.int32)
        alphas.append(tab_ref[tidx])

    bv = b_ref[...]
    starts = [jnp.int32(0)]
    for g in range(1, _B):
        starts.append(jnp.sum((bv < g).astype(jnp.int32)))

    a = jnp.full(shape, alphas[0], jnp.float32)
    for g in range(1, _B):
        a = jnp.where(m >= starts[g] * 4, alphas[g], a)

    xv = x_ref[...]
    q_off = (1.0 - a) * 0.25
    q_diag = a + q_off
    dx = q_diag * xv
    ox = q_off * xv
    o1 = _group_rot(ox, 1)
    o2 = _group_rot(ox, 2)
    o3 = _group_rot(ox, 3)

    def term(j):
        return jnp.where(c == j, dx,
                         jnp.where(c == ((j - 1) % 4), o1,
                                   jnp.where(c == ((j - 2) % 4), o2, o3)))

    p = (term(0) + term(1)) + (term(2) + term(3))
    prob_ref[...] = p

    z1 = p + _group_rot(p, 1)
    z2 = z1 + _group_rot(z1, 2)
    s = jnp.where(c == 0, z2,
                  jnp.where(c == 1, _group_rot(z2, 3),
                            jnp.where(c == 2, _group_rot(z2, 2),
                                      _group_rot(z2, 1))))

    z = gum + jnp.log(p / s + 1e-12)

    win = None
    for k in (1, 2, 3):
        zk = _group_rot(z, k)
        cmp_k = (z > zk) | ((c + k < 4) & (z == zk))
        win = cmp_k if win is None else (win & cmp_k)
    noise_ref[...] = jnp.where(win, jnp.float32(1.0), jnp.float32(0.0))


def kernel(x, batch, t_int):
    tab = jnp.asarray(_ALPHAS_BAR)
    xp = x.reshape(_ROWS, 128)
    prob_p, noise_p = pl.pallas_call(
        _fused_kernel,
        out_shape=[
            jax.ShapeDtypeStruct((_ROWS, 128), jnp.float32),
            jax.ShapeDtypeStruct((_ROWS, 128), jnp.float32),
        ],
        in_specs=[
            pl.BlockSpec(memory_space=pltpu.SMEM),
            pl.BlockSpec(memory_space=pltpu.SMEM),
            pl.BlockSpec(memory_space=pltpu.VMEM),
            pl.BlockSpec(memory_space=pltpu.VMEM),
        ],
        out_specs=[
            pl.BlockSpec(memory_space=pltpu.VMEM),
            pl.BlockSpec(memory_space=pltpu.VMEM),
        ],
    )(t_int, tab, batch.reshape(128, 128), xp)
    return prob_p.reshape(_N, _K), noise_p.reshape(_N, _K)


# speedup vs baseline: 8.0842x; 8.0842x over previous
import numpy as np
import jax
import jax.numpy as jnp
from jax import lax
from jax.experimental import pallas as pl
from jax.experimental.pallas import tpu as pltpu

_TIMESTEPS = 500
_K = 4
_B = 16
_N = 16384
_CHUNKS = 8
_LPC = _N // _CHUNKS


def _cosine_alphas_bar_host(timesteps, s=0.008):
    steps = timesteps + 2
    x = np.linspace(0, steps, steps)
    ac = np.cos(0.5 * np.pi * ((x / steps) + s) / (1 + s)) ** 2
    ac = ac / ac[0]
    alphas_step = ac[1:] / ac[:-1]
    betas = np.clip(1.0 - alphas_step, 0.0, 0.9999)
    return np.exp(np.cumsum(np.log(1.0 - betas))).astype(np.float32)


_ALPHAS_BAR = _cosine_alphas_bar_host(_TIMESTEPS)
_TINY = np.float32(np.finfo(np.float32).tiny)


def _chunk(t, alphas, b_ref, x_ref, prob_ref, noise_ref):
    shape = (_K, _LPC)
    sl = pl.ds(t * _LPC, _LPC)
    lane = lax.broadcasted_iota(jnp.int32, shape, 1) + t * _LPC
    ri = lax.broadcasted_iota(jnp.int32, shape, 0)
    m = lane * 4 + ri

    def rotl(v, d):
        return lax.shift_left(v, jnp.int32(d)) | lax.shift_right_logical(
            v, jnp.int32(32 - d))

    def rounds(x0, x1, rots):
        for r in rots:
            x0 = x0 + x1
            x1 = rotl(x1, r)
            x1 = x0 ^ x1
        return x0, x1

    ks0 = jnp.int32(0)
    ks1 = jnp.int32(1)
    ks2 = jnp.int32(0x1BD11BDB)
    r13 = (13, 15, 26, 6)
    r17 = (17, 29, 16, 24)
    x0 = jnp.zeros(shape, jnp.int32) + ks0
    x1 = m + ks1
    x0, x1 = rounds(x0, x1, r13)
    x0, x1 = x0 + ks1, x1 + (ks2 + jnp.int32(1))
    x0, x1 = rounds(x0, x1, r17)
    x0, x1 = x0 + ks2, x1 + (ks0 + jnp.int32(2))
    x0, x1 = rounds(x0, x1, r13)
    x0, x1 = x0 + ks0, x1 + (ks1 + jnp.int32(3))
    x0, x1 = rounds(x0, x1, r17)
    x0, x1 = x0 + ks1, x1 + (ks2 + jnp.int32(4))
    x0, x1 = rounds(x0, x1, r13)
    x0, x1 = x0 + ks2, x1 + (ks0 + jnp.int32(5))
    bits = x0 ^ x1

    fb = lax.shift_right_logical(bits, jnp.int32(9)) | jnp.int32(0x3F800000)
    f = lax.bitcast_convert_type(fb, jnp.float32) - jnp.float32(1.0)
    u = jnp.maximum(jnp.float32(_TINY),
                    f * (jnp.float32(1.0) - _TINY) + _TINY)
    gum = -jnp.log(-jnp.log(u))

    b = b_ref[:, sl]
    a = jnp.zeros(b.shape, jnp.float32)
    for g in range(_B):
        a = jnp.where(b == g, alphas[g], a)

    q_off = (1.0 - a) * 0.25
    q_diag = a + q_off

    xt = x_ref[:, sl]
    diag_t = q_diag * xt
    off_t = q_off * xt

    def term(j):
        return jnp.where(ri == j, diag_t[j:j + 1, :], off_t[j:j + 1, :])

    p = (term(0) + term(1)) + (term(2) + term(3))
    prob_ref[:, sl] = p

    s = (p[0:1, :] + p[1:2, :]) + (p[2:3, :] + p[3:4, :])
    z = gum + jnp.log(p / s + 1e-12)

    best = z[0:1, :]
    idx = jnp.zeros(b.shape, jnp.int32)
    for i in range(1, _K):
        zi = z[i:i + 1, :]
        better = zi > best
        idx = jnp.where(better, i, idx)
        best = jnp.where(better, zi, best)
    noise_ref[:, sl] = jnp.where(ri == idx, jnp.float32(1.0), jnp.float32(0.0))


def _fused_kernel(t_ref, tab_ref, b_ref, x_ref, prob_ref, noise_ref):
    alphas = []
    for g in range(_B):
        tg = t_ref[g, 0]
        tf = tg.astype(jnp.float32) / jnp.float32(_TIMESTEPS)
        tidx = (tf * jnp.float32(_TIMESTEPS) + jnp.float32(0.5)).astype(jnp.int32)
        alphas.append(tab_ref[tidx])

    for t in range(_CHUNKS):
        _chunk(t, alphas, b_ref, x_ref, prob_ref, noise_ref)


def kernel(x, batch, t_int):
    tab = jnp.asarray(_ALPHAS_BAR)
    xt = x.T
    b2 = batch[None, :]
    prob_t, noise_t = pl.pallas_call(
        _fused_kernel,
        out_shape=[
            jax.ShapeDtypeStruct((_K, _N), jnp.float32),
            jax.ShapeDtypeStruct((_K, _N), jnp.float32),
        ],
        in_specs=[
            pl.BlockSpec(memory_space=pltpu.SMEM),
            pl.BlockSpec(memory_space=pltpu.SMEM),
            pl.BlockSpec(memory_space=pltpu.VMEM),
            pl.BlockSpec(memory_space=pltpu.VMEM),
        ],
        out_specs=[
            pl.BlockSpec(memory_space=pltpu.VMEM),
            pl.BlockSpec(memory_space=pltpu.VMEM),
        ],
    )(t_int, tab, b2, xt)
    return prob_t.T, noise_t.T


# speedup vs baseline: 9.5620x; 1.1828x over previous
import numpy as np
import jax
import jax.numpy as jnp
from jax import lax
from jax.experimental import pallas as pl
from jax.experimental.pallas import tpu as pltpu

_TIMESTEPS = 500
_K = 4
_B = 16
_N = 16384
_CHUNKS = 8
_LPC = _N // _CHUNKS


def _cosine_alphas_bar_host(timesteps, s=0.008):
    steps = timesteps + 2
    x = np.linspace(0, steps, steps)
    ac = np.cos(0.5 * np.pi * ((x / steps) + s) / (1 + s)) ** 2
    ac = ac / ac[0]
    alphas_step = ac[1:] / ac[:-1]
    betas = np.clip(1.0 - alphas_step, 0.0, 0.9999)
    return np.exp(np.cumsum(np.log(1.0 - betas))).astype(np.float32)


_ALPHAS_BAR = _cosine_alphas_bar_host(_TIMESTEPS)
_TINY = np.float32(np.finfo(np.float32).tiny)


def _dense_gumbel(blk):
    shape = (8, _LPC)
    off = 2 * blk * _LPC
    s8 = lax.broadcasted_iota(jnp.int32, shape, 0)
    l8 = lax.broadcasted_iota(jnp.int32, shape, 1)
    m = 4 * off + 4 * l8 + (s8 & 3) + jnp.where(
        s8 >= 4, jnp.int32(4 * _LPC), jnp.int32(0))

    def rotl(v, d):
        return lax.shift_left(v, jnp.int32(d)) | lax.shift_right_logical(
            v, jnp.int32(32 - d))

    def rounds(x0, x1, rots):
        for r in rots:
            x0 = x0 + x1
            x1 = rotl(x1, r)
            x1 = x0 ^ x1
        return x0, x1

    ks0 = jnp.int32(0)
    ks1 = jnp.int32(1)
    ks2 = jnp.int32(0x1BD11BDB)
    r13 = (13, 15, 26, 6)
    r17 = (17, 29, 16, 24)
    x0 = jnp.zeros(shape, jnp.int32) + ks0
    x1 = m + ks1
    x0, x1 = rounds(x0, x1, r13)
    x0, x1 = x0 + ks1, x1 + (ks2 + jnp.int32(1))
    x0, x1 = rounds(x0, x1, r17)
    x0, x1 = x0 + ks2, x1 + (ks0 + jnp.int32(2))
    x0, x1 = rounds(x0, x1, r13)
    x0, x1 = x0 + ks0, x1 + (ks1 + jnp.int32(3))
    x0, x1 = rounds(x0, x1, r17)
    x0, x1 = x0 + ks1, x1 + (ks2 + jnp.int32(4))
    x0, x1 = rounds(x0, x1, r13)
    x0, x1 = x0 + ks2, x1 + (ks0 + jnp.int32(5))
    bits = x0 ^ x1

    fb = lax.shift_right_logical(bits, jnp.int32(9)) | jnp.int32(0x3F800000)
    f = lax.bitcast_convert_type(fb, jnp.float32) - jnp.float32(1.0)
    u = jnp.maximum(jnp.float32(_TINY),
                    f * (jnp.float32(1.0) - _TINY) + _TINY)
    return -jnp.log(-jnp.log(u))


def _chunk(t, gum, alphas, b_ref, x_ref, prob_ref, noise_ref):
    sl = pl.ds(t * _LPC, _LPC)
    ri = lax.broadcasted_iota(jnp.int32, (_K, _LPC), 0)

    b = b_ref[:, sl]
    a = jnp.zeros(b.shape, jnp.float32)
    for g in range(_B):
        a = jnp.where(b == g, alphas[g], a)

    q_off = (1.0 - a) * 0.25
    q_diag = a + q_off

    xt = x_ref[:, sl]
    diag_t = q_diag * xt
    off_t = q_off * xt

    def term(j):
        return jnp.where(ri == j, diag_t[j:j + 1, :], off_t[j:j + 1, :])

    p = (term(0) + term(1)) + (term(2) + term(3))
    prob_ref[:, sl] = p

    s = (p[0:1, :] + p[1:2, :]) + (p[2:3, :] + p[3:4, :])
    z = gum + jnp.log(p / s + 1e-12)

    best = z[0:1, :]
    idx = jnp.zeros(b.shape, jnp.int32)
    for i in range(1, _K):
        zi = z[i:i + 1, :]
        better = zi > best
        idx = jnp.where(better, i, idx)
        best = jnp.where(better, zi, best)
    noise_ref[:, sl] = jnp.where(ri == idx, jnp.float32(1.0), jnp.float32(0.0))


def _fused_kernel(t_ref, tab_ref, b_ref, x_ref, prob_ref, noise_ref):
    alphas = []
    for g in range(_B):
        tg = t_ref[g, 0]
        tf = tg.astype(jnp.float32) / jnp.float32(_TIMESTEPS)
        tidx = (tf * jnp.float32(_TIMESTEPS) + jnp.float32(0.5)).astype(jnp.int32)
        alphas.append(tab_ref[tidx])

    for blk in range(_CHUNKS // 2):
        gum8 = _dense_gumbel(blk)
        _chunk(2 * blk, gum8[0:4, :], alphas, b_ref, x_ref,
               prob_ref, noise_ref)
        _chunk(2 * blk + 1, gum8[4:8, :], alphas, b_ref, x_ref,
               prob_ref, noise_ref)


def kernel(x, batch, t_int):
    tab = jnp.asarray(_ALPHAS_BAR)
    xt = x.T
    b2 = batch[None, :]
    prob_t, noise_t = pl.pallas_call(
        _fused_kernel,
        out_shape=[
            jax.ShapeDtypeStruct((_K, _N), jnp.float32),
            jax.ShapeDtypeStruct((_K, _N), jnp.float32),
        ],
        in_specs=[
            pl.BlockSpec(memory_space=pltpu.SMEM),
            pl.BlockSpec(memory_space=pltpu.SMEM),
            pl.BlockSpec(memory_space=pltpu.VMEM),
            pl.BlockSpec(memory_space=pltpu.VMEM),
        ],
        out_specs=[
            pl.BlockSpec(memory_space=pltpu.VMEM),
            pl.BlockSpec(memory_space=pltpu.VMEM),
        ],
    )(t_int, tab, b2, xt)
    return prob_t.T, noise_t.T


# speedup vs baseline: 9.6883x; 1.0132x over previous
import numpy as np
import jax
import jax.numpy as jnp
from jax import lax
from jax.experimental import pallas as pl
from jax.experimental.pallas import tpu as pltpu

_TIMESTEPS = 500
_K = 4
_B = 16
_N = 16384
_PAIRS = 4
_LPC = _N // (2 * _PAIRS)


def _cosine_alphas_bar_host(timesteps, s=0.008):
    steps = timesteps + 2
    x = np.linspace(0, steps, steps)
    ac = np.cos(0.5 * np.pi * ((x / steps) + s) / (1 + s)) ** 2
    ac = ac / ac[0]
    alphas_step = ac[1:] / ac[:-1]
    betas = np.clip(1.0 - alphas_step, 0.0, 0.9999)
    return np.exp(np.cumsum(np.log(1.0 - betas))).astype(np.float32)


_ALPHAS_BAR = _cosine_alphas_bar_host(_TIMESTEPS)
_TINY = np.float32(np.finfo(np.float32).tiny)


def _group_rot(c, v, d):
    if d == 0:
        return v
    fwd = jnp.roll(v, -d, 0)
    bwd = jnp.roll(v, 4 - d, 0)
    return jnp.where(c < 4 - d, fwd, bwd)


def _pair(blk, alphas, b_ref, x_ref, prob_ref, noise_ref):
    shape = (8, _LPC)
    off = 2 * blk * _LPC
    sl_a = pl.ds(off, _LPC)
    sl_b = pl.ds(off + _LPC, _LPC)
    s8 = lax.broadcasted_iota(jnp.int32, shape, 0)
    l8 = lax.broadcasted_iota(jnp.int32, shape, 1)
    c = s8 & 3
    m = 4 * off + 4 * l8 + c + jnp.where(
        s8 >= 4, jnp.int32(4 * _LPC), jnp.int32(0))

    def rotl(v, d):
        return lax.shift_left(v, jnp.int32(d)) | lax.shift_right_logical(
            v, jnp.int32(32 - d))

    def rounds(x0, x1, rots):
        for r in rots:
            x0 = x0 + x1
            x1 = rotl(x1, r)
            x1 = x0 ^ x1
        return x0, x1

    ks0 = jnp.int32(0)
    ks1 = jnp.int32(1)
    ks2 = jnp.int32(0x1BD11BDB)
    r13 = (13, 15, 26, 6)
    r17 = (17, 29, 16, 24)
    x0 = jnp.zeros(shape, jnp.int32) + ks0
    x1 = m + ks1
    x0, x1 = rounds(x0, x1, r13)
    x0, x1 = x0 + ks1, x1 + (ks2 + jnp.int32(1))
    x0, x1 = rounds(x0, x1, r17)
    x0, x1 = x0 + ks2, x1 + (ks0 + jnp.int32(2))
    x0, x1 = rounds(x0, x1, r13)
    x0, x1 = x0 + ks0, x1 + (ks1 + jnp.int32(3))
    x0, x1 = rounds(x0, x1, r17)
    x0, x1 = x0 + ks1, x1 + (ks2 + jnp.int32(4))
    x0, x1 = rounds(x0, x1, r13)
    x0, x1 = x0 + ks2, x1 + (ks0 + jnp.int32(5))
    bits = x0 ^ x1

    fb = lax.shift_right_logical(bits, jnp.int32(9)) | jnp.int32(0x3F800000)
    f = lax.bitcast_convert_type(fb, jnp.float32) - jnp.float32(1.0)
    u = jnp.maximum(jnp.float32(_TINY),
                    f * (jnp.float32(1.0) - _TINY) + _TINY)
    gum = -jnp.log(-jnp.log(u))

    xv = jnp.concatenate([x_ref[:, sl_a], x_ref[:, sl_b]], axis=0)
    b8 = jnp.concatenate(
        [jnp.broadcast_to(b_ref[:, sl_a], (_K, _LPC)),
         jnp.broadcast_to(b_ref[:, sl_b], (_K, _LPC))], axis=0)

    a = jnp.zeros(shape, jnp.float32)
    for g in range(_B):
        a = jnp.where(b8 == g, alphas[g], a)

    q_off = (1.0 - a) * 0.25
    q_diag = a + q_off
    dx = q_diag * xv
    ox = q_off * xv
    o1 = _group_rot(c, ox, 1)
    o2 = _group_rot(c, ox, 2)
    o3 = _group_rot(c, ox, 3)
    codd = (c & 1) == 1
    pa = jnp.where(codd, o3 + dx, dx + o1)
    pb = jnp.where(codd, o1 + o2, o2 + o3)
    p = pa + pb
    prob_ref[:, sl_a] = p[0:4, :]
    prob_ref[:, sl_b] = p[4:8, :]

    e1 = p + _group_rot(c, p, 1)
    q1 = jnp.where(codd, _group_rot(c, e1, 3), e1)
    q2 = _group_rot(c, q1, 2)
    s = q1 + q2

    z = gum + jnp.log(p / s + 1e-12)

    win = None
    for k in (1, 2, 3):
        zk = _group_rot(c, z, k)
        cmp_k = (z > zk) | ((c + k < 4) & (z == zk))
        win = cmp_k if win is None else (win & cmp_k)
    noise = jnp.where(win, jnp.float32(1.0), jnp.float32(0.0))
    noise_ref[:, sl_a] = noise[0:4, :]
    noise_ref[:, sl_b] = noise[4:8, :]


def _fused_kernel(t_ref, tab_ref, b_ref, x_ref, prob_ref, noise_ref):
    alphas = []
    for g in range(_B):
        tg = t_ref[g, 0]
        tf = tg.astype(jnp.float32) / jnp.float32(_TIMESTEPS)
        tidx = (tf * jnp.float32(_TIMESTEPS) + jnp.float32(0.5)).astype(jnp.int32)
        alphas.append(tab_ref[tidx])

    for blk in range(_PAIRS):
        _pair(blk, alphas, b_ref, x_ref, prob_ref, noise_ref)


def kernel(x, batch, t_int):
    tab = jnp.asarray(_ALPHAS_BAR)
    xt = x.T
    b2 = batch[None, :]
    prob_t, noise_t = pl.pallas_call(
        _fused_kernel,
        out_shape=[
            jax.ShapeDtypeStruct((_K, _N), jnp.float32),
            jax.ShapeDtypeStruct((_K, _N), jnp.float32),
        ],
        in_specs=[
            pl.BlockSpec(memory_space=pltpu.SMEM),
            pl.BlockSpec(memory_space=pltpu.SMEM),
            pl.BlockSpec(memory_space=pltpu.VMEM),
            pl.BlockSpec(memory_space=pltpu.VMEM),
        ],
        out_specs=[
            pl.BlockSpec(memory_space=pltpu.VMEM),
            pl.BlockSpec(memory_space=pltpu.VMEM),
        ],
    )(t_int, tab, b2, xt)
    return prob_t.T, noise_t.T
